# baseline (speedup 1.0000x reference)
import functools

import jax
import jax.numpy as jnp
from jax import lax
from jax.experimental import pallas as pl
from jax.experimental.pallas import tpu as pltpu
from jax.experimental.pallas import tpu_sc as plsc

_N = 5000
_B = 256
_NB = 20
_NPAD = _B * _NB
_IOU_TH = 0.4
_SCORE_TH = 0.3

_D = 16
_NW = 32
_RPW = _NPAD // _NW
_CH = _RPW // 2


@functools.partial(
    pl.kernel,
    out_type=jax.ShapeDtypeStruct((_NPAD, _D), jnp.float32),
    mesh=plsc.VectorSubcoreMesh(core_axis_name="c", subcore_axis_name="s"),
    compiler_params=pltpu.CompilerParams(use_tc_tiling_on_sc=False),
    scratch_types=[
        pltpu.VMEM((_RPW,), jnp.int32),
        pltpu.VMEM((_RPW, _D), jnp.float32),
        pltpu.SemaphoreType.DMA,
    ],
)
def _sc_gather(feat_hbm, idx_hbm, out_hbm, idx_v, rows_v, sem):
    wid = lax.axis_index("s") * 2 + lax.axis_index("c")
    base = wid * _RPW
    pltpu.sync_copy(idx_hbm.at[pl.ds(base, _RPW)], idx_v)
    cp0 = pltpu.async_copy(feat_hbm.at[idx_v.at[pl.ds(0, _CH)]],
                           rows_v.at[pl.ds(0, _CH)], sem)
    cp1 = pltpu.async_copy(feat_hbm.at[idx_v.at[pl.ds(_CH, _CH)]],
                           rows_v.at[pl.ds(_CH, _CH)], sem)
    cp0.wait()
    cp1.wait()
    pltpu.sync_copy(rows_v, out_hbm.at[pl.ds(base, _RPW)])


def _iou_gt(rows_bcast, cm):
    x1a, y1a, x2a, y2a, aa = rows_bcast
    x1b, y1b, x2b, y2b, ab = (cm[0:1, :], cm[1:2, :], cm[2:3, :],
                              cm[3:4, :], cm[4:5, :])
    w = jnp.maximum(jnp.minimum(x2a, x2b) - jnp.maximum(x1a, x1b), 0.0)
    h = jnp.maximum(jnp.minimum(y2a, y2b) - jnp.maximum(y1a, y1b), 0.0)
    inter = w * h
    union = aa + ab - inter
    iou = inter / (union + 1e-6)
    return (iou > _IOU_TH).astype(jnp.float32)


def _nms_body(rows_ref, cols_ref, keep_ref):
    keep_ref[...] = jnp.ones((_NB, 1, _B), jnp.float32)
    ii = jax.lax.broadcasted_iota(jnp.int32, (_B, _B), 0)
    jj = jax.lax.broadcasted_iota(jnp.int32, (_B, _B), 1)
    tri = (ii < jj).astype(jnp.float32)

    def block_step(k, carry):
        rk = rows_ref[k]
        rb = tuple(jnp.broadcast_to(rk[:, c:c + 1], (_B, _B))
                   for c in range(5))
        m_kk = _iou_gt(rb, cols_ref[k]) * tri
        k_init = keep_ref[k]

        def fp_cond(c):
            return c[1]

        def fp_body(c):
            kv, _ = c
            sup = jnp.dot(kv, m_kk, preferred_element_type=jnp.float32)
            kn = k_init * (sup == 0.0).astype(jnp.float32)
            return kn, jnp.sum(jnp.abs(kn - kv)) > 0.0

        kv, _ = jax.lax.while_loop(fp_cond, fp_body,
                                   (k_init, jnp.asarray(True)))
        keep_ref[k] = kv

        def later8(t, inner):
            m0 = k + 1 + 8 * t
            for d in range(8):
                m = jnp.minimum(m0 + d, _NB - 1)
                sup = jnp.dot(kv, _iou_gt(rb, cols_ref[m]),
                              preferred_element_type=jnp.float32)
                keep_ref[m] = keep_ref[m] * (sup == 0.0).astype(jnp.float32)
            return inner

        cnt = _NB - 1 - k
        jax.lax.fori_loop(0, (cnt + 7) // 8, later8, 0)
        return carry

    jax.lax.fori_loop(0, _NB, block_step, 0)

    def thresh(k, carry):
        keep_ref[k] = keep_ref[k] * (
            cols_ref[k][5:6, :] > _SCORE_TH).astype(jnp.float32)
        return carry

    jax.lax.fori_loop(0, _NB, thresh, 0)


def kernel(boxes, scores):
    order = jnp.argsort(-scores)
    area = (boxes[:, 2] - boxes[:, 0]) * (boxes[:, 3] - boxes[:, 1])
    feat_u = jnp.concatenate(
        [boxes, area[:, None], scores[:, None],
         jnp.zeros((_N, _D - 6), jnp.float32)], axis=1)
    feat_u = jnp.concatenate(
        [feat_u, jnp.zeros((_NPAD - _N, _D), jnp.float32)], axis=0)
    order_pad = jnp.concatenate(
        [order.astype(jnp.int32), jnp.arange(_N, _NPAD, dtype=jnp.int32)])
    feat16 = _sc_gather(feat_u, order_pad)
    b = feat16[:_N, 0:4]
    s = feat16[:_N, 5]
    feat = feat16[:, 0:8]
    rows = feat.reshape(_NB, _B, 8)
    cols = jnp.transpose(rows, (0, 2, 1))
    keep = pl.pallas_call(
        _nms_body,
        out_shape=jax.ShapeDtypeStruct((_NB, 1, _B), jnp.float32),
    )(rows, cols)
    kf = keep.reshape(_NPAD)[:_N]
    return jnp.concatenate([b * kf[:, None], (s * kf)[:, None]], axis=1)


# speedup vs baseline: 1.0542x; 1.0542x over previous
import functools

import jax
import jax.numpy as jnp
from jax import lax
from jax.experimental import pallas as pl
from jax.experimental.pallas import tpu as pltpu
from jax.experimental.pallas import tpu_sc as plsc

_N = 5000
_B = 256
_NB = 20
_NPAD = _B * _NB
_IOU_TH = 0.4
_SCORE_TH = 0.3

_D = 16
_NW = 32
_RPW = _NPAD // _NW
_CH = _RPW // 2


@functools.partial(
    pl.kernel,
    out_type=jax.ShapeDtypeStruct((_NPAD, _D), jnp.float32),
    mesh=plsc.VectorSubcoreMesh(core_axis_name="c", subcore_axis_name="s"),
    compiler_params=pltpu.CompilerParams(use_tc_tiling_on_sc=False),
    scratch_types=[
        pltpu.VMEM((_RPW,), jnp.int32),
        pltpu.VMEM((_RPW, _D), jnp.float32),
        pltpu.SemaphoreType.DMA,
    ],
)
def _sc_gather(feat_hbm, idx_hbm, out_hbm, idx_v, rows_v, sem):
    wid = lax.axis_index("s") * 2 + lax.axis_index("c")
    base = wid * _RPW
    pltpu.sync_copy(idx_hbm.at[pl.ds(base, _RPW)], idx_v)
    cp0 = pltpu.async_copy(feat_hbm.at[idx_v.at[pl.ds(0, _CH)]],
                           rows_v.at[pl.ds(0, _CH)], sem)
    cp1 = pltpu.async_copy(feat_hbm.at[idx_v.at[pl.ds(_CH, _CH)]],
                           rows_v.at[pl.ds(_CH, _CH)], sem)
    cp0.wait()
    cp1.wait()
    pltpu.sync_copy(rows_v, out_hbm.at[pl.ds(base, _RPW)])


def _iou_gt(rows_bcast, cm):
    x1a, y1a, x2a, y2a, aa = rows_bcast
    x1b, y1b, x2b, y2b, ab = (cm[0:1, :], cm[1:2, :], cm[2:3, :],
                              cm[3:4, :], cm[4:5, :])
    w = jnp.maximum(jnp.minimum(x2a, x2b) - jnp.maximum(x1a, x1b), 0.0)
    h = jnp.maximum(jnp.minimum(y2a, y2b) - jnp.maximum(y1a, y1b), 0.0)
    inter = w * h
    union = aa + ab - inter
    iou = inter / (union + 1e-6)
    return (iou > _IOU_TH).astype(jnp.float32)


def _nms_body(rows_ref, cols_ref, keep_ref):
    keep_ref[...] = jnp.ones((_NB, 1, _B), jnp.float32)
    ii = jax.lax.broadcasted_iota(jnp.int32, (_B, _B), 0)
    jj = jax.lax.broadcasted_iota(jnp.int32, (_B, _B), 1)
    tri = (ii < jj).astype(jnp.float32)

    def block_step(k, carry):
        rk = rows_ref[k]
        rb = tuple(jnp.broadcast_to(rk[:, c:c + 1], (_B, _B))
                   for c in range(5))
        m_kk = _iou_gt(rb, cols_ref[k]) * tri
        k_init = keep_ref[k]

        def fp_cond(c):
            return c[1]

        def fp_body(c):
            kv, _ = c
            sup = jnp.dot(kv, m_kk, preferred_element_type=jnp.float32)
            kn = k_init * (sup == 0.0).astype(jnp.float32)
            return kn, jnp.sum(jnp.abs(kn - kv)) > 0.0

        kv, _ = jax.lax.while_loop(fp_cond, fp_body,
                                   (k_init, jnp.asarray(True)))
        keep_ref[k] = kv

        def suppress(m):
            sup = jnp.dot(kv, _iou_gt(rb, cols_ref[m]),
                          preferred_element_type=jnp.float32)
            keep_ref[m] = keep_ref[m] * (sup == 0.0).astype(jnp.float32)

        def later4(t, inner):
            m0 = k + 1 + 4 * t
            for d in range(4):
                suppress(m0 + d)
            return inner

        def later1(m, inner):
            suppress(m)
            return inner

        cnt = _NB - 1 - k
        full = cnt // 4
        jax.lax.fori_loop(0, full, later4, 0)
        jax.lax.fori_loop(k + 1 + 4 * full, _NB, later1, 0)
        return carry

    jax.lax.fori_loop(0, _NB, block_step, 0)

    def thresh(k, carry):
        keep_ref[k] = keep_ref[k] * (
            cols_ref[k][5:6, :] > _SCORE_TH).astype(jnp.float32)
        return carry

    jax.lax.fori_loop(0, _NB, thresh, 0)


def kernel(boxes, scores):
    order = jnp.argsort(-scores)
    area = (boxes[:, 2] - boxes[:, 0]) * (boxes[:, 3] - boxes[:, 1])
    feat_u = jnp.concatenate(
        [boxes, area[:, None], scores[:, None],
         jnp.zeros((_N, _D - 6), jnp.float32)], axis=1)
    feat_u = jnp.concatenate(
        [feat_u, jnp.zeros((_NPAD - _N, _D), jnp.float32)], axis=0)
    order_pad = jnp.concatenate(
        [order.astype(jnp.int32), jnp.arange(_N, _NPAD, dtype=jnp.int32)])
    feat16 = _sc_gather(feat_u, order_pad)
    b = feat16[:_N, 0:4]
    s = feat16[:_N, 5]
    feat = feat16[:, 0:8]
    rows = feat.reshape(_NB, _B, 8)
    cols = jnp.transpose(rows, (0, 2, 1))
    keep = pl.pallas_call(
        _nms_body,
        out_shape=jax.ShapeDtypeStruct((_NB, 1, _B), jnp.float32),
    )(rows, cols)
    kf = keep.reshape(_NPAD)[:_N]
    return jnp.concatenate([b * kf[:, None], (s * kf)[:, None]], axis=1)


# speedup vs baseline: 1.1643x; 1.1045x over previous
import functools

import jax
import jax.numpy as jnp
from jax import lax
from jax.experimental import pallas as pl
from jax.experimental.pallas import tpu as pltpu
from jax.experimental.pallas import tpu_sc as plsc

_N = 5000
_B = 512
_NB = 10
_NPAD = _B * _NB
_IOU_TH = 0.4
_SCORE_TH = 0.3

_D = 16
_NW = 32
_RPW = _NPAD // _NW
_CH = _RPW // 2


@functools.partial(
    pl.kernel,
    out_type=jax.ShapeDtypeStruct((_NPAD, _D), jnp.float32),
    mesh=plsc.VectorSubcoreMesh(core_axis_name="c", subcore_axis_name="s"),
    compiler_params=pltpu.CompilerParams(use_tc_tiling_on_sc=False),
    scratch_types=[
        pltpu.VMEM((_RPW,), jnp.int32),
        pltpu.VMEM((_RPW, _D), jnp.float32),
        pltpu.SemaphoreType.DMA,
    ],
)
def _sc_gather(feat_hbm, idx_hbm, out_hbm, idx_v, rows_v, sem):
    wid = lax.axis_index("s") * 2 + lax.axis_index("c")
    base = wid * _RPW
    pltpu.sync_copy(idx_hbm.at[pl.ds(base, _RPW)], idx_v)
    cp0 = pltpu.async_copy(feat_hbm.at[idx_v.at[pl.ds(0, _CH)]],
                           rows_v.at[pl.ds(0, _CH)], sem)
    cp1 = pltpu.async_copy(feat_hbm.at[idx_v.at[pl.ds(_CH, _CH)]],
                           rows_v.at[pl.ds(_CH, _CH)], sem)
    cp0.wait()
    cp1.wait()
    pltpu.sync_copy(rows_v, out_hbm.at[pl.ds(base, _RPW)])


def _iou_gt(rows_bcast, cm):
    x1a, y1a, x2a, y2a, aa = rows_bcast
    x1b, y1b, x2b, y2b, ab = (cm[0:1, :], cm[1:2, :], cm[2:3, :],
                              cm[3:4, :], cm[4:5, :])
    w = jnp.maximum(jnp.minimum(x2a, x2b) - jnp.maximum(x1a, x1b), 0.0)
    h = jnp.maximum(jnp.minimum(y2a, y2b) - jnp.maximum(y1a, y1b), 0.0)
    inter = w * h
    union = aa + ab - inter
    iou = inter / (union + 1e-6)
    return (iou > _IOU_TH).astype(jnp.float32)


def _nms_body(rows_ref, cols_ref, keep_ref):
    keep_ref[...] = jnp.ones((_NB, 1, _B), jnp.float32)
    ii = jax.lax.broadcasted_iota(jnp.int32, (_B, _B), 0)
    jj = jax.lax.broadcasted_iota(jnp.int32, (_B, _B), 1)
    tri = (ii < jj).astype(jnp.float32)

    def block_step(k, carry):
        rk = rows_ref[k]
        rb = tuple(jnp.broadcast_to(rk[:, c:c + 1], (_B, _B))
                   for c in range(5))
        m_kk = _iou_gt(rb, cols_ref[k]) * tri
        k_init = keep_ref[k]

        def fp_cond(c):
            return c[1]

        def fp_body(c):
            kv, _ = c
            sup = jnp.dot(kv, m_kk, preferred_element_type=jnp.float32)
            kn = k_init * (sup == 0.0).astype(jnp.float32)
            return kn, jnp.sum(jnp.abs(kn - kv)) > 0.0

        kv, _ = jax.lax.while_loop(fp_cond, fp_body,
                                   (k_init, jnp.asarray(True)))
        keep_ref[k] = kv

        def suppress(m):
            sup = jnp.dot(kv, _iou_gt(rb, cols_ref[m]),
                          preferred_element_type=jnp.float32)
            keep_ref[m] = keep_ref[m] * (sup == 0.0).astype(jnp.float32)

        def later4(t, inner):
            m0 = k + 1 + 4 * t
            for d in range(4):
                suppress(m0 + d)
            return inner

        def later1(m, inner):
            suppress(m)
            return inner

        cnt = _NB - 1 - k
        full = cnt // 4
        jax.lax.fori_loop(0, full, later4, 0)
        jax.lax.fori_loop(k + 1 + 4 * full, _NB, later1, 0)
        return carry

    jax.lax.fori_loop(0, _NB, block_step, 0)

    def thresh(k, carry):
        keep_ref[k] = keep_ref[k] * (
            cols_ref[k][5:6, :] > _SCORE_TH).astype(jnp.float32)
        return carry

    jax.lax.fori_loop(0, _NB, thresh, 0)


def kernel(boxes, scores):
    order = jnp.argsort(-scores)
    area = (boxes[:, 2] - boxes[:, 0]) * (boxes[:, 3] - boxes[:, 1])
    feat_u = jnp.concatenate(
        [boxes, area[:, None], scores[:, None],
         jnp.zeros((_N, _D - 6), jnp.float32)], axis=1)
    feat_u = jnp.concatenate(
        [feat_u, jnp.zeros((_NPAD - _N, _D), jnp.float32)], axis=0)
    order_pad = jnp.concatenate(
        [order.astype(jnp.int32), jnp.arange(_N, _NPAD, dtype=jnp.int32)])
    feat16 = _sc_gather(feat_u, order_pad)
    b = feat16[:_N, 0:4]
    s = feat16[:_N, 5]
    feat = feat16[:, 0:8]
    rows = feat.reshape(_NB, _B, 8)
    cols = jnp.transpose(rows, (0, 2, 1))
    keep = pl.pallas_call(
        _nms_body,
        out_shape=jax.ShapeDtypeStruct((_NB, 1, _B), jnp.float32),
    )(rows, cols)
    kf = keep.reshape(_NPAD)[:_N]
    return jnp.concatenate([b * kf[:, None], (s * kf)[:, None]], axis=1)


# speedup vs baseline: 1.1818x; 1.0150x over previous
import functools

import jax
import jax.numpy as jnp
from jax import lax
from jax.experimental import pallas as pl
from jax.experimental.pallas import tpu as pltpu
from jax.experimental.pallas import tpu_sc as plsc

_N = 5000
_B = 512
_NB = 10
_NPAD = _B * _NB
_IOU_TH = 0.4
_SCORE_TH = 0.3

_D = 16
_NW = 32
_RPW = _NPAD // _NW
_CH = _RPW // 2


@functools.partial(
    pl.kernel,
    out_type=jax.ShapeDtypeStruct((_NPAD, _D), jnp.float32),
    mesh=plsc.VectorSubcoreMesh(core_axis_name="c", subcore_axis_name="s"),
    compiler_params=pltpu.CompilerParams(use_tc_tiling_on_sc=False),
    scratch_types=[
        pltpu.VMEM((_RPW,), jnp.int32),
        pltpu.VMEM((_RPW, _D), jnp.float32),
        pltpu.SemaphoreType.DMA,
    ],
)
def _sc_gather(feat_hbm, idx_hbm, out_hbm, idx_v, rows_v, sem):
    wid = lax.axis_index("s") * 2 + lax.axis_index("c")
    base = wid * _RPW
    pltpu.sync_copy(idx_hbm.at[pl.ds(base, _RPW)], idx_v)
    cp0 = pltpu.async_copy(feat_hbm.at[idx_v.at[pl.ds(0, _CH)]],
                           rows_v.at[pl.ds(0, _CH)], sem)
    cp1 = pltpu.async_copy(feat_hbm.at[idx_v.at[pl.ds(_CH, _CH)]],
                           rows_v.at[pl.ds(_CH, _CH)], sem)
    cp0.wait()
    cp1.wait()
    pltpu.sync_copy(rows_v, out_hbm.at[pl.ds(base, _RPW)])


def _iou_gt(rows_bcast, cm):
    x1a, y1a, x2a, y2a, aa = rows_bcast
    x1b, y1b, x2b, y2b, ab = (cm[0:1, :], cm[1:2, :], cm[2:3, :],
                              cm[3:4, :], cm[4:5, :])
    w = jnp.maximum(jnp.minimum(x2a, x2b) - jnp.maximum(x1a, x1b), 0.0)
    h = jnp.maximum(jnp.minimum(y2a, y2b) - jnp.maximum(y1a, y1b), 0.0)
    inter = w * h
    union = aa + ab - inter
    iou = inter / (union + 1e-6)
    return (iou > _IOU_TH).astype(jnp.float32)


def _nms_body(rows_ref, cols_ref, keep_ref):
    keep_ref[...] = jnp.ones((_NB, 1, _B), jnp.float32)
    ii = jax.lax.broadcasted_iota(jnp.int32, (_B, _B), 0)
    jj = jax.lax.broadcasted_iota(jnp.int32, (_B, _B), 1)
    tri = (ii < jj).astype(jnp.float32)

    def block_step(k, carry):
        rk = rows_ref[k]
        rb = tuple(jnp.broadcast_to(rk[:, c:c + 1], (_B, _B))
                   for c in range(5))
        m_kk = _iou_gt(rb, cols_ref[k]) * tri
        k_init = keep_ref[k]

        def fp_cond(c):
            return c[1]

        def fp_body(c):
            kv, _ = c
            sup = jnp.dot(kv, m_kk, preferred_element_type=jnp.float32)
            k1 = k_init * (sup == 0.0).astype(jnp.float32)
            sup2 = jnp.dot(k1, m_kk, preferred_element_type=jnp.float32)
            k2 = k_init * (sup2 == 0.0).astype(jnp.float32)
            return k2, jnp.sum(jnp.abs(k2 - k1)) > 0.0

        kv, _ = jax.lax.while_loop(fp_cond, fp_body,
                                   (k_init, jnp.asarray(True)))
        keep_ref[k] = kv

        def suppress(m):
            sup = jnp.dot(kv, _iou_gt(rb, cols_ref[m]),
                          preferred_element_type=jnp.float32)
            keep_ref[m] = keep_ref[m] * (sup == 0.0).astype(jnp.float32)

        def later4(t, inner):
            m0 = k + 1 + 4 * t
            for d in range(4):
                suppress(m0 + d)
            return inner

        def later1(m, inner):
            suppress(m)
            return inner

        cnt = _NB - 1 - k
        full = cnt // 4
        jax.lax.fori_loop(0, full, later4, 0)
        jax.lax.fori_loop(k + 1 + 4 * full, _NB, later1, 0)
        return carry

    jax.lax.fori_loop(0, _NB, block_step, 0)

    def thresh(k, carry):
        keep_ref[k] = keep_ref[k] * (
            cols_ref[k][5:6, :] > _SCORE_TH).astype(jnp.float32)
        return carry

    jax.lax.fori_loop(0, _NB, thresh, 0)


def kernel(boxes, scores):
    order = jnp.argsort(-scores)
    area = (boxes[:, 2] - boxes[:, 0]) * (boxes[:, 3] - boxes[:, 1])
    feat_u = jnp.concatenate(
        [boxes, area[:, None], scores[:, None],
         jnp.zeros((_N, _D - 6), jnp.float32)], axis=1)
    feat_u = jnp.concatenate(
        [feat_u, jnp.zeros((_NPAD - _N, _D), jnp.float32)], axis=0)
    order_pad = jnp.concatenate(
        [order.astype(jnp.int32), jnp.arange(_N, _NPAD, dtype=jnp.int32)])
    feat16 = _sc_gather(feat_u, order_pad)
    b = feat16[:_N, 0:4]
    s = feat16[:_N, 5]
    feat = feat16[:, 0:8]
    rows = feat.reshape(_NB, _B, 8)
    cols = jnp.transpose(rows, (0, 2, 1))
    keep = pl.pallas_call(
        _nms_body,
        out_shape=jax.ShapeDtypeStruct((_NB, 1, _B), jnp.float32),
    )(rows, cols)
    kf = keep.reshape(_NPAD)[:_N]
    return jnp.concatenate([b * kf[:, None], (s * kf)[:, None]], axis=1)
